# baseline (speedup 1.0000x reference)
import functools

import jax
import jax.numpy as jnp
from jax import lax
from jax.experimental import pallas as pl
from jax.experimental.pallas import tpu as pltpu
from jax.experimental.pallas import tpu_sc as plsc

B = 4096
EMB = 32
HID = 64
NC = 2
NS = 16
NW = NC * NS
BPW = B // NW
WAVE = 4
NBANKS = 4
NWAVES = BPW // WAVE
NGENRE = 1000
RB = 1024
GRID = B // RB


def _iota16():
    return lax.iota(jnp.int32, 16)


def _splat16(x):
    return jnp.full((16,), x, jnp.int32)


def _wg_ids(idx_v, w):
    return idx_v[pl.ds(w * WAVE, 16)]


def _window_gather(idx_v, table_t, out_t, win_v, ebuf, semw, base,
                   bias1d=None, bwin=None, bbuf=None):
    iota = _iota16()

    def fire(w, bank):
        idx16 = _wg_ids(idx_v, w)
        for t in range(WAVE):
            r = idx16[t]
            j = pl.multiple_of((r >> 7) * 128, 128)
            pltpu.async_copy(table_t.at[:, pl.ds(j, 128)],
                             win_v.at[bank, t], semw.at[bank])
            if bias1d is not None:
                pltpu.async_copy(bias1d.at[pl.ds(j, 128)],
                                 bwin.at[bank, t], semw.at[bank])

    def drain_extract(w, bank):
        idx16 = _wg_ids(idx_v, w)
        for t in range(WAVE):
            r = idx16[t]
            j = pl.multiple_of((r >> 7) * 128, 128)
            pltpu.make_async_copy(table_t.at[:, pl.ds(j, 128)],
                                  win_v.at[bank, t], semw.at[bank]).wait()
            if bias1d is not None:
                pltpu.make_async_copy(bias1d.at[pl.ds(j, 128)],
                                      bwin.at[bank, t], semw.at[bank]).wait()
        for t in range(WAVE):
            r = idx16[t]
            cc = r & 127
            col = _splat16(w * WAVE + t)
            lo = plsc.load_gather(win_v, [_splat16(bank), _splat16(t), iota,
                                          _splat16(cc)])
            hi = plsc.load_gather(win_v, [_splat16(bank), _splat16(t),
                                          iota + 16, _splat16(cc)])
            plsc.store_scatter(ebuf, [iota, col], lo)
            plsc.store_scatter(ebuf, [iota + 16, col], hi)

    def bias_extract(w, bank):
        sel = jnp.where(iota < WAVE, iota, 0)
        idx8 = plsc.load_gather(idx_v, [_splat16(w * WAVE) + sel])
        bv = plsc.load_gather(bwin, [_splat16(bank), sel, idx8 & 127])
        plsc.store_scatter(bbuf, [_splat16(w * WAVE) + sel], bv,
                           mask=iota < WAVE)

    def body(k, _):
        w0 = NBANKS * k
        for b in range(NBANKS):
            drain_extract(w0 + b, b)
            if bias1d is not None:
                bias_extract(w0 + b, b)
            fire(w0 + b + NBANKS, b)
        return 0

    for b in range(NBANKS):
        fire(b, b)
    lax.fori_loop(0, NWAVES // NBANKS - 1, body, 0, unroll=False)
    w0 = NWAVES - NBANKS
    for b in range(NBANKS):
        drain_extract(w0 + b, b)
        if bias1d is not None:
            bias_extract(w0 + b, b)
    obase = pl.multiple_of(base, 128)
    pltpu.sync_copy(ebuf, out_t.at[:, pl.ds(obase, BPW)])


def _sc_gather_body(uid, mid, gid, ut_t, mt_t, gt_t, bias1d,
                    uo_t, mo_t, go_t, bo,
                    idx_v, win_v, ebuf, gvm, bwin, bbuf, semw, semg):
    c = lax.axis_index("c")
    s = lax.axis_index("s")
    wid = s * NC + c
    base = wid * BPW
    iota = _iota16()
    obase = pl.multiple_of(base, 128)

    gstage = pltpu.async_copy(gt_t, gvm, semg)

    pltpu.sync_copy(uid.at[pl.ds(base, BPW)], idx_v.at[pl.ds(0, BPW)])
    _window_gather(idx_v, ut_t, uo_t, win_v, ebuf, semw, base)

    pltpu.sync_copy(mid.at[pl.ds(base, BPW)], idx_v.at[pl.ds(0, BPW)])
    _window_gather(idx_v, mt_t, mo_t, win_v, ebuf, semw, base,
                   bias1d=bias1d, bwin=bwin, bbuf=bbuf)
    pltpu.sync_copy(bbuf, bo.at[pl.ds(obase, BPW)])

    gstage.wait()
    pltpu.sync_copy(gid.at[pl.ds(base, BPW)], idx_v.at[pl.ds(0, BPW)])

    def g_wave(w, _):
        idx16 = idx_v[pl.ds(w * 16, 16)]
        col = _splat16(w * 16) + iota
        for e in range(EMB):
            vals = plsc.load_gather(gvm, [_splat16(e), idx16])
            plsc.store_scatter(ebuf, [_splat16(e), col], vals)
        return 0

    lax.fori_loop(0, BPW // 16, g_wave, 0, unroll=False)
    pltpu.sync_copy(ebuf, go_t.at[:, pl.ds(obase, BPW)])


@functools.cache
def _sc_gather():
    return pl.kernel(
        _sc_gather_body,
        out_type=(
            jax.ShapeDtypeStruct((EMB, B), jnp.float32),
            jax.ShapeDtypeStruct((EMB, B), jnp.float32),
            jax.ShapeDtypeStruct((EMB, B), jnp.float32),
            jax.ShapeDtypeStruct((B,), jnp.float32),
        ),
        mesh=plsc.VectorSubcoreMesh(core_axis_name="c", subcore_axis_name="s"),
        compiler_params=pltpu.CompilerParams(use_tc_tiling_on_sc=True,
                                             needs_layout_passes=False),
        scratch_types=(
            pltpu.VMEM((BPW + 16,), jnp.int32),
            pltpu.VMEM((NBANKS, WAVE, EMB, 128), jnp.float32),
            pltpu.VMEM((EMB, BPW), jnp.float32),
            pltpu.VMEM((EMB, NGENRE), jnp.float32),
            pltpu.VMEM((NBANKS, WAVE, 128), jnp.float32),
            pltpu.VMEM((BPW,), jnp.float32),
            pltpu.SemaphoreType.DMA((NBANKS,)),
            pltpu.SemaphoreType.DMA,
        ),
    )


def _tc_body(ue_t, me_t, ge_t, brow, w1, b1, w2r, b2, out_ref, a_s, b_s):
    i = pl.program_id(0)

    @pl.when(i == 0)
    def _():
        u = ue_t[...]
        m = me_t[...]
        g = ge_t[...]
        dn = (((0,), (0,)), ((), ()))
        h = (lax.dot_general(u, w1[0:EMB, :], dn,
                             preferred_element_type=jnp.float32)
             + lax.dot_general(m, w1[EMB:2 * EMB, :], dn,
                               preferred_element_type=jnp.float32)
             + lax.dot_general(g, w1[2 * EMB:3 * EMB, :], dn,
                               preferred_element_type=jnp.float32)
             + b1[...])
        h = jnp.maximum(h, 0.0)
        a_s[...] = jnp.sum(h * w2r[...], axis=1, keepdims=True) + b2[...]
        wide = jnp.sum(u + m + g, axis=0, keepdims=True)
        b_s[...] = wide + brow[...]

    out_ref[...] = a_s[pl.ds(i * RB, RB), :] + b_s[...]


def _tc_fused(ue_t, me_t, ge_t, brow, w1, b1r, w2r, b2r):
    full = lambda i: (0, 0)
    return pl.pallas_call(
        _tc_body,
        grid=(GRID,),
        in_specs=[
            pl.BlockSpec((EMB, B), full),
            pl.BlockSpec((EMB, B), full),
            pl.BlockSpec((EMB, B), full),
            pl.BlockSpec((1, B), full),
            pl.BlockSpec((3 * EMB, HID), full),
            pl.BlockSpec((1, HID), full),
            pl.BlockSpec((1, HID), full),
            pl.BlockSpec((1, 1), full),
        ],
        out_specs=pl.BlockSpec((RB, B), lambda i: (i, 0)),
        out_shape=jax.ShapeDtypeStruct((B, B), jnp.float32),
        scratch_shapes=[
            pltpu.VMEM((B, 1), jnp.float32),
            pltpu.VMEM((1, B), jnp.float32),
        ],
    )(ue_t, me_t, ge_t, brow, w1, b1r, w2r, b2r)


def kernel(user_ids, movie_ids, genre_ids, user_table, movie_table,
           genre_table, movie_bias_table, W1, b1, W2, b2):
    uid = user_ids.astype(jnp.int32)
    mid = movie_ids.astype(jnp.int32)
    gid = genre_ids.astype(jnp.int32)
    ut_t = user_table.T
    mt_t = movie_table.T
    gt_t = genre_table.T
    bias1d = movie_bias_table.reshape(-1)
    ue_t, me_t, ge_t, bvals = _sc_gather()(uid, mid, gid, ut_t, mt_t, gt_t,
                                           bias1d)
    brow = bvals.reshape(1, B)
    b1r = b1.reshape(1, HID)
    w2r = W2.reshape(1, HID)
    b2r = b2.reshape(1, 1)
    return _tc_fused(ue_t, me_t, ge_t, brow, W1, b1r, w2r, b2r)


# speedup vs baseline: 1.0069x; 1.0069x over previous
import functools

import jax
import jax.numpy as jnp
from jax import lax
from jax.experimental import pallas as pl
from jax.experimental.pallas import tpu as pltpu
from jax.experimental.pallas import tpu_sc as plsc

B = 4096
EMB = 32
HID = 64
NC = 2
NS = 16
NW = NC * NS
BPW = B // NW
WAVE = 4
NBANKS = 4
NWAVES = BPW // WAVE
NGENRE = 1000
RB = 512
GRID = B // RB


def _iota16():
    return lax.iota(jnp.int32, 16)


def _splat16(x):
    return jnp.full((16,), x, jnp.int32)


def _wg_ids(idx_v, w):
    return idx_v[pl.ds(w * WAVE, 16)]


def _window_gather(idx_v, table_t, out_t, win_v, ebuf, semw, base,
                   bias1d=None, bwin=None, bbuf=None):
    iota = _iota16()

    def fire(w, bank):
        idx16 = _wg_ids(idx_v, w)
        for t in range(WAVE):
            r = idx16[t]
            j = pl.multiple_of((r >> 7) * 128, 128)
            pltpu.async_copy(table_t.at[:, pl.ds(j, 128)],
                             win_v.at[bank, t], semw.at[bank])
            if bias1d is not None:
                pltpu.async_copy(bias1d.at[pl.ds(j, 128)],
                                 bwin.at[bank, t], semw.at[bank])

    def drain_extract(w, bank):
        idx16 = _wg_ids(idx_v, w)
        for t in range(WAVE):
            r = idx16[t]
            j = pl.multiple_of((r >> 7) * 128, 128)
            pltpu.make_async_copy(table_t.at[:, pl.ds(j, 128)],
                                  win_v.at[bank, t], semw.at[bank]).wait()
            if bias1d is not None:
                pltpu.make_async_copy(bias1d.at[pl.ds(j, 128)],
                                      bwin.at[bank, t], semw.at[bank]).wait()
        for t in range(WAVE):
            r = idx16[t]
            cc = r & 127
            col = _splat16(w * WAVE + t)
            lo = plsc.load_gather(win_v, [_splat16(bank), _splat16(t), iota,
                                          _splat16(cc)])
            hi = plsc.load_gather(win_v, [_splat16(bank), _splat16(t),
                                          iota + 16, _splat16(cc)])
            plsc.store_scatter(ebuf, [iota, col], lo)
            plsc.store_scatter(ebuf, [iota + 16, col], hi)

    def bias_extract(w, bank):
        sel = jnp.where(iota < WAVE, iota, 0)
        idx8 = plsc.load_gather(idx_v, [_splat16(w * WAVE) + sel])
        bv = plsc.load_gather(bwin, [_splat16(bank), sel, idx8 & 127])
        plsc.store_scatter(bbuf, [_splat16(w * WAVE) + sel], bv,
                           mask=iota < WAVE)

    def body(k, _):
        w0 = NBANKS * k
        for b in range(NBANKS):
            drain_extract(w0 + b, b)
            if bias1d is not None:
                bias_extract(w0 + b, b)
            fire(w0 + b + NBANKS, b)
        return 0

    for b in range(NBANKS):
        fire(b, b)
    lax.fori_loop(0, NWAVES // NBANKS - 1, body, 0, unroll=False)
    w0 = NWAVES - NBANKS
    for b in range(NBANKS):
        drain_extract(w0 + b, b)
        if bias1d is not None:
            bias_extract(w0 + b, b)
    obase = pl.multiple_of(base, 128)
    pltpu.sync_copy(ebuf, out_t.at[:, pl.ds(obase, BPW)])


def _sc_gather_body(uid, mid, gid, ut_t, mt_t, gt_t, bias1d,
                    uo_t, mo_t, go_t, bo,
                    idx_v, win_v, ebuf, gvm, bwin, bbuf, semw, semg):
    c = lax.axis_index("c")
    s = lax.axis_index("s")
    wid = s * NC + c
    base = wid * BPW
    iota = _iota16()
    obase = pl.multiple_of(base, 128)

    gstage = pltpu.async_copy(gt_t, gvm, semg)

    pltpu.sync_copy(uid.at[pl.ds(base, BPW)], idx_v.at[pl.ds(0, BPW)])
    _window_gather(idx_v, ut_t, uo_t, win_v, ebuf, semw, base)

    pltpu.sync_copy(mid.at[pl.ds(base, BPW)], idx_v.at[pl.ds(0, BPW)])
    _window_gather(idx_v, mt_t, mo_t, win_v, ebuf, semw, base,
                   bias1d=bias1d, bwin=bwin, bbuf=bbuf)
    pltpu.sync_copy(bbuf, bo.at[pl.ds(obase, BPW)])

    gstage.wait()
    pltpu.sync_copy(gid.at[pl.ds(base, BPW)], idx_v.at[pl.ds(0, BPW)])

    def g_wave(w, _):
        idx16 = idx_v[pl.ds(w * 16, 16)]
        col = _splat16(w * 16) + iota
        for e in range(EMB):
            vals = plsc.load_gather(gvm, [_splat16(e), idx16])
            plsc.store_scatter(ebuf, [_splat16(e), col], vals)
        return 0

    lax.fori_loop(0, BPW // 16, g_wave, 0, unroll=False)
    pltpu.sync_copy(ebuf, go_t.at[:, pl.ds(obase, BPW)])


@functools.cache
def _sc_gather():
    return pl.kernel(
        _sc_gather_body,
        out_type=(
            jax.ShapeDtypeStruct((EMB, B), jnp.float32),
            jax.ShapeDtypeStruct((EMB, B), jnp.float32),
            jax.ShapeDtypeStruct((EMB, B), jnp.float32),
            jax.ShapeDtypeStruct((B,), jnp.float32),
        ),
        mesh=plsc.VectorSubcoreMesh(core_axis_name="c", subcore_axis_name="s"),
        compiler_params=pltpu.CompilerParams(use_tc_tiling_on_sc=True,
                                             needs_layout_passes=False),
        scratch_types=(
            pltpu.VMEM((BPW + 16,), jnp.int32),
            pltpu.VMEM((NBANKS, WAVE, EMB, 128), jnp.float32),
            pltpu.VMEM((EMB, BPW), jnp.float32),
            pltpu.VMEM((EMB, NGENRE), jnp.float32),
            pltpu.VMEM((NBANKS, WAVE, 128), jnp.float32),
            pltpu.VMEM((BPW,), jnp.float32),
            pltpu.SemaphoreType.DMA((NBANKS,)),
            pltpu.SemaphoreType.DMA,
        ),
    )


def _tc_body(ue_t, me_t, ge_t, brow, w1, b1, w2r, b2, out_ref, a_s, b_s):
    i = pl.program_id(0)

    @pl.when(i == 0)
    def _():
        u = ue_t[...]
        m = me_t[...]
        g = ge_t[...]
        dn = (((0,), (0,)), ((), ()))
        h = (lax.dot_general(u, w1[0:EMB, :], dn,
                             preferred_element_type=jnp.float32)
             + lax.dot_general(m, w1[EMB:2 * EMB, :], dn,
                               preferred_element_type=jnp.float32)
             + lax.dot_general(g, w1[2 * EMB:3 * EMB, :], dn,
                               preferred_element_type=jnp.float32)
             + b1[...])
        h = jnp.maximum(h, 0.0)
        a_s[...] = jnp.sum(h * w2r[...], axis=1, keepdims=True) + b2[...]
        wide = jnp.sum(u + m + g, axis=0, keepdims=True)
        b_s[...] = wide + brow[...]

    out_ref[...] = a_s[pl.ds(i * RB, RB), :] + b_s[...]


def _tc_fused(ue_t, me_t, ge_t, brow, w1, b1r, w2r, b2r):
    full = lambda i: (0, 0)
    return pl.pallas_call(
        _tc_body,
        grid=(GRID,),
        in_specs=[
            pl.BlockSpec((EMB, B), full),
            pl.BlockSpec((EMB, B), full),
            pl.BlockSpec((EMB, B), full),
            pl.BlockSpec((1, B), full),
            pl.BlockSpec((3 * EMB, HID), full),
            pl.BlockSpec((1, HID), full),
            pl.BlockSpec((1, HID), full),
            pl.BlockSpec((1, 1), full),
        ],
        out_specs=pl.BlockSpec((RB, B), lambda i: (i, 0)),
        out_shape=jax.ShapeDtypeStruct((B, B), jnp.float32),
        scratch_shapes=[
            pltpu.VMEM((B, 1), jnp.float32),
            pltpu.VMEM((1, B), jnp.float32),
        ],
    )(ue_t, me_t, ge_t, brow, w1, b1r, w2r, b2r)


def kernel(user_ids, movie_ids, genre_ids, user_table, movie_table,
           genre_table, movie_bias_table, W1, b1, W2, b2):
    uid = user_ids.astype(jnp.int32)
    mid = movie_ids.astype(jnp.int32)
    gid = genre_ids.astype(jnp.int32)
    ut_t = user_table.T
    mt_t = movie_table.T
    gt_t = genre_table.T
    bias1d = movie_bias_table.reshape(-1)
    ue_t, me_t, ge_t, bvals = _sc_gather()(uid, mid, gid, ut_t, mt_t, gt_t,
                                           bias1d)
    brow = bvals.reshape(1, B)
    b1r = b1.reshape(1, HID)
    w2r = W2.reshape(1, HID)
    b2r = b2.reshape(1, 1)
    return _tc_fused(ue_t, me_t, ge_t, brow, W1, b1r, w2r, b2r)
